# Initial kernel scaffold; baseline (speedup 1.0000x reference)
import jax
import jax.numpy as jnp
from jax.experimental import pallas as pl


def kernel(x, edge_weight, edge_index):
    raise NotImplementedError("write your pallas kernel here")


# speedup vs baseline: 9.6999x; 9.6999x over previous
import functools

import jax
import jax.numpy as jnp
from jax import lax
from jax.experimental import pallas as pl
from jax.experimental.pallas import tpu as pltpu
from jax.experimental.pallas import tpu_sc as plsc

NC = 2
NS = 16
NW = NC * NS
W = 128
CH = 16
LANES = 16


def _scale_rows(w_v, j, rows_buf, d):
    groups = W // LANES

    def grp(g, carry):
        base = g * LANES
        w16 = w_v[j, pl.ds(base, LANES)]
        for e in range(LANES):
            w_s = w16[e]
            for q in range(d // LANES):
                sl = pl.ds(q * LANES, LANES)
                rows_buf[base + e, sl] = rows_buf[base + e, sl] * w_s
        return carry

    lax.fori_loop(0, groups, grp, 0)


def _make_sc_kernel(n, d, nw):
    rpt = (n // NS) // 8 * 8
    rem = n - NS * rpt
    nch = nw // CH
    mesh = plsc.VectorSubcoreMesh(core_axis_name="c", subcore_axis_name="s")

    @functools.partial(
        pl.kernel,
        out_type=jax.ShapeDtypeStruct((NC, n, d), jnp.float32),
        mesh=mesh,
        scratch_types=[
            pltpu.VMEM((CH, W), jnp.int32),
            pltpu.VMEM((CH, W), jnp.int32),
            pltpu.VMEM((CH, W), jnp.float32),
            pltpu.VMEM((W, d), jnp.float32),
            pltpu.VMEM((W, d), jnp.float32),
            pltpu.VMEM_SHARED((n, d), jnp.float32),
            pltpu.SemaphoreType.DMA,
            pltpu.SemaphoreType.DMA,
        ],
    )
    def sc_kernel(x_hbm, src_hbm, dst_hbm, w_hbm, zeros_hbm, out_hbm,
                  src_v, dst_v, w_v, rows_a, rows_b, acc, sem_a, sem_b):
        c = lax.axis_index("c")
        s = lax.axis_index("s")
        wid = s * NC + c

        pltpu.sync_copy(zeros_hbm.at[pl.ds(0, rpt)], acc.at[pl.ds(s * rpt, rpt)])
        if rem:
            @pl.when(s == NS - 1)
            def _():
                pltpu.sync_copy(zeros_hbm.at[pl.ds(0, rem)],
                                acc.at[pl.ds(NS * rpt, rem)])
        plsc.subcore_barrier()

        def gather_start(j, buf, sem):
            pltpu.make_async_copy(x_hbm.at[src_v.at[j]], buf, sem).start()

        def gather_wait(j, buf, sem):
            pltpu.make_async_copy(x_hbm.at[src_v.at[j]], buf, sem).wait()

        def do_window(j, rows_buf):
            _scale_rows(w_v, j, rows_buf, d)
            pltpu.sync_copy(rows_buf, acc.at[dst_v.at[j]], add=True)

        def chunk(ch, carry):
            sl = pl.ds(ch * CH, CH)
            pltpu.sync_copy(src_hbm.at[wid, sl], src_v)
            pltpu.sync_copy(dst_hbm.at[wid, sl], dst_v)
            pltpu.sync_copy(w_hbm.at[wid, sl], w_v)
            gather_start(0, rows_a, sem_a)

            def step(i, carry):
                j0 = i * 2
                j1 = j0 + 1
                gather_wait(j0, rows_a, sem_a)
                gather_start(j1, rows_b, sem_b)
                do_window(j0, rows_a)
                gather_wait(j1, rows_b, sem_b)

                @pl.when(j1 + 1 < CH)
                def _():
                    gather_start(j1 + 1, rows_a, sem_a)

                do_window(j1, rows_b)
                return carry

            lax.fori_loop(0, CH // 2, step, 0)
            return carry

        lax.fori_loop(0, nch, chunk, 0)

        plsc.subcore_barrier()
        pltpu.sync_copy(acc.at[pl.ds(s * rpt, rpt)],
                        out_hbm.at[c, pl.ds(s * rpt, rpt)])
        if rem:
            @pl.when(s == NS - 1)
            def _():
                pltpu.sync_copy(acc.at[pl.ds(NS * rpt, rem)],
                                out_hbm.at[c, pl.ds(NS * rpt, rem)])

    return sc_kernel


def _combine_body(p_ref, o_ref):
    o_ref[...] = p_ref[0] + p_ref[1]


def kernel(x, edge_weight, edge_index):
    n, d = x.shape
    e = edge_weight.shape[0]

    epw = -(-e // (NW * CH * W)) * CH * W
    nw = epw // W
    pad = NW * epw - e
    pad_idx = (jnp.arange(pad, dtype=jnp.int32) * 97) % n
    src = jnp.concatenate([edge_index[0].astype(jnp.int32), pad_idx])
    dst = jnp.concatenate([edge_index[1].astype(jnp.int32), pad_idx])
    w = jnp.concatenate([edge_weight, jnp.zeros((pad,), edge_weight.dtype)])
    src = src.reshape(NW, nw, W)
    dst = dst.reshape(NW, nw, W)
    w = w.reshape(NW, nw, W)
    zeros = jnp.zeros((n // NS // 8 * 8, d), jnp.float32)

    partials = _make_sc_kernel(n, d, nw)(x, src, dst, w, zeros)

    rows_blk = 1000
    return pl.pallas_call(
        _combine_body,
        out_shape=jax.ShapeDtypeStruct((n, d), jnp.float32),
        grid=(n // rows_blk,),
        in_specs=[pl.BlockSpec((NC, rows_blk, d), lambda i: (0, i, 0))],
        out_specs=pl.BlockSpec((rows_blk, d), lambda i: (i, 0)),
    )(partials)
